# Initial kernel scaffold; baseline (speedup 1.0000x reference)
import jax
import jax.numpy as jnp
from jax.experimental import pallas as pl


def kernel(hidden_states, Wq, bq, Wk, bk, W1, b1, W2, b2):
    raise NotImplementedError("write your pallas kernel here")


# speedup vs baseline: 1.4283x; 1.4283x over previous
import jax
import jax.numpy as jnp
from jax.experimental import pallas as pl
from jax.experimental.pallas import tpu as pltpu

_D = 768
_H = _D // 2
_EPS = 1e-8
_THRESH = 0.5
_T = 512


def _chunker_kernel(x_ref, wh_ref, bcat_ref, w2_ref, b2_ref,
                    f_ref, m_ref, klast_ref):
    j = pl.program_id(1)
    x = x_ref[0]
    xh = x.astype(jnp.bfloat16)
    y = jnp.dot(xh, wh_ref[...], preferred_element_type=jnp.float32)
    y = y + bcat_ref[...]
    q = y[:, :_D]
    k = y[:, _D:2 * _D]
    h = jnp.maximum(y[:, 2 * _D:], 0.0)

    hb = h.astype(jnp.bfloat16).astype(jnp.float32)
    wb = w2_ref[...].astype(jnp.bfloat16).astype(jnp.float32)
    z = jnp.sum(hb * wb, axis=1, keepdims=True) + b2_ref[...]
    r = jax.nn.sigmoid(z)

    nq = jnp.sqrt(jnp.sum(q * q, axis=1, keepdims=True))
    nk = jnp.sqrt(jnp.sum(k * k, axis=1, keepdims=True))

    k_prev = klast_ref[...]
    kshift = pltpu.roll(k, 1, 0)
    row_d = jax.lax.broadcasted_iota(jnp.int32, kshift.shape, 0)
    kshift = jnp.where(row_d == 0, jnp.broadcast_to(k_prev, kshift.shape),
                       kshift)
    nk0 = jnp.sqrt(jnp.sum(k_prev * k_prev, axis=1, keepdims=True))
    row_1 = jax.lax.broadcasted_iota(jnp.int32, nk.shape, 0)
    nkshift = jnp.where(row_1 == 0, nk0, pltpu.roll(nk, 1, 0))

    dots = jnp.sum(q * kshift, axis=1, keepdims=True)
    sim = dots / (jnp.maximum(nq, _EPS) * jnp.maximum(nkshift, _EPS))
    bp = 0.5 * (1.0 - sim)
    bp = jnp.where(jnp.logical_and(j == 0, row_1 == 0), 1.0, bp)

    final = 0.7 * bp + 0.3 * r
    f_ref[0] = final
    m_ref[0] = (final > _THRESH).astype(jnp.int32)
    klast_ref[...] = k[-1:, :]


def kernel(hidden_states, Wq, bq, Wk, bk, W1, b1, W2, b2):
    B, S, D = hidden_states.shape
    wcat = jnp.concatenate([Wq, Wk, W1], axis=1)
    wh = wcat.astype(jnp.bfloat16)
    bcat = jnp.concatenate([bq, bk, b1])[None, :]
    w2 = W2.reshape(1, _H)
    b2r = b2.reshape(1, 1)
    n_tiles = S // _T

    f, m = pl.pallas_call(
        _chunker_kernel,
        grid=(B, n_tiles),
        in_specs=[
            pl.BlockSpec((1, _T, D), lambda b, j: (b, j, 0)),
            pl.BlockSpec((D, 2 * _D + _H), lambda b, j: (0, 0)),
            pl.BlockSpec((1, 2 * _D + _H), lambda b, j: (0, 0)),
            pl.BlockSpec((1, _H), lambda b, j: (0, 0)),
            pl.BlockSpec((1, 1), lambda b, j: (0, 0)),
        ],
        out_specs=[
            pl.BlockSpec((1, _T, 1), lambda b, j: (b, j, 0)),
            pl.BlockSpec((1, _T, 1), lambda b, j: (b, j, 0)),
        ],
        out_shape=[
            jax.ShapeDtypeStruct((B, S, 1), jnp.float32),
            jax.ShapeDtypeStruct((B, S, 1), jnp.int32),
        ],
        scratch_shapes=[pltpu.VMEM((1, D), jnp.float32)],
        compiler_params=pltpu.CompilerParams(
            dimension_semantics=("arbitrary", "arbitrary"),
        ),
    )(hidden_states, wh, bcat, w2, b2r)
    return f[..., 0], m[..., 0]


# speedup vs baseline: 1.5060x; 1.0544x over previous
import jax
import jax.numpy as jnp
from jax.experimental import pallas as pl
from jax.experimental.pallas import tpu as pltpu

_D = 768
_H = _D // 2
_EPS = 1e-8
_THRESH = 0.5
_T = 1024


def _chunker_kernel(x_ref, wh_ref, bcat_ref, w2_ref, b2_ref,
                    f_ref, m_ref, klast_ref):
    j = pl.program_id(1)
    x = x_ref[0]
    xh = x.astype(jnp.bfloat16)
    y = jnp.dot(xh, wh_ref[...], preferred_element_type=jnp.float32)
    y = y + bcat_ref[...]
    q = y[:, :_D]
    k = y[:, _D:2 * _D]
    h = jnp.maximum(y[:, 2 * _D:], 0.0)

    hb = h.astype(jnp.bfloat16).astype(jnp.float32)
    wb = w2_ref[...].astype(jnp.bfloat16).astype(jnp.float32)
    z = jnp.sum(hb * wb, axis=1, keepdims=True) + b2_ref[...]
    r = jax.nn.sigmoid(z)

    nq = jnp.sqrt(jnp.sum(q * q, axis=1, keepdims=True))
    nk = jnp.sqrt(jnp.sum(k * k, axis=1, keepdims=True))

    k_prev = klast_ref[...]
    kshift = pltpu.roll(k, 1, 0)
    row_d = jax.lax.broadcasted_iota(jnp.int32, kshift.shape, 0)
    kshift = jnp.where(row_d == 0, jnp.broadcast_to(k_prev, kshift.shape),
                       kshift)
    nk0 = jnp.sqrt(jnp.sum(k_prev * k_prev, axis=1, keepdims=True))
    row_1 = jax.lax.broadcasted_iota(jnp.int32, nk.shape, 0)
    nkshift = jnp.where(row_1 == 0, nk0, pltpu.roll(nk, 1, 0))

    dots = jnp.sum(q * kshift, axis=1, keepdims=True)
    sim = dots / (jnp.maximum(nq, _EPS) * jnp.maximum(nkshift, _EPS))
    bp = 0.5 * (1.0 - sim)
    bp = jnp.where(jnp.logical_and(j == 0, row_1 == 0), 1.0, bp)

    final = 0.7 * bp + 0.3 * r
    f_ref[0] = final
    m_ref[0] = (final > _THRESH).astype(jnp.int32)
    klast_ref[...] = k[-1:, :]


def kernel(hidden_states, Wq, bq, Wk, bk, W1, b1, W2, b2):
    B, S, D = hidden_states.shape
    wcat = jnp.concatenate([Wq, Wk, W1], axis=1)
    wh = wcat.astype(jnp.bfloat16)
    bcat = jnp.concatenate([bq, bk, b1])[None, :]
    w2 = W2.reshape(1, _H)
    b2r = b2.reshape(1, 1)
    n_tiles = S // _T

    f, m = pl.pallas_call(
        _chunker_kernel,
        grid=(B, n_tiles),
        in_specs=[
            pl.BlockSpec((1, _T, D), lambda b, j: (b, j, 0)),
            pl.BlockSpec((D, 2 * _D + _H), lambda b, j: (0, 0)),
            pl.BlockSpec((1, 2 * _D + _H), lambda b, j: (0, 0)),
            pl.BlockSpec((1, _H), lambda b, j: (0, 0)),
            pl.BlockSpec((1, 1), lambda b, j: (0, 0)),
        ],
        out_specs=[
            pl.BlockSpec((1, _T, 1), lambda b, j: (b, j, 0)),
            pl.BlockSpec((1, _T, 1), lambda b, j: (b, j, 0)),
        ],
        out_shape=[
            jax.ShapeDtypeStruct((B, S, 1), jnp.float32),
            jax.ShapeDtypeStruct((B, S, 1), jnp.int32),
        ],
        scratch_shapes=[pltpu.VMEM((1, D), jnp.float32)],
        compiler_params=pltpu.CompilerParams(
            dimension_semantics=("arbitrary", "arbitrary"),
        ),
    )(hidden_states, wh, bcat, w2, b2r)
    return f[..., 0], m[..., 0]
